# baseline (speedup 1.0000x reference)
import functools

import jax
import jax.numpy as jnp
from jax import lax
from jax.experimental import pallas as pl
from jax.experimental.pallas import tpu as pltpu
from jax.experimental.pallas import tpu_sc as plsc

_L = 16

_MESH = plsc.VectorSubcoreMesh(core_axis_name="c", subcore_axis_name="s")


@functools.partial(
    pl.kernel,
    mesh=_MESH,
    out_type=jax.ShapeDtypeStruct((32, 96), jnp.float32),
    scratch_types=[
        pltpu.VMEM((96,), jnp.float32),
        pltpu.VMEM((64,), jnp.int32),
        pltpu.VMEM((96,), jnp.float32),
    ],
)
def _sc_segsum_outer(data_hbm, idx_hbm, out_hbm, data_v, idx_v, row_v):
    c = lax.axis_index("c")
    s = lax.axis_index("s")
    wid = s * 2 + c

    pltpu.sync_copy(data_hbm, data_v)
    pltpu.sync_copy(idx_hbm, idx_v)

    zeros = jnp.zeros((_L,), jnp.float32)
    acc = zeros
    for k in range(4):
        v = data_v[pl.ds(k * _L, _L)]
        ix = idx_v[pl.ds(k * _L, _L)]
        acc = acc + jnp.where(ix == wid, v, zeros)
    seg = acc[0]
    for i in range(1, _L):
        seg = seg + acc[i]

    row_v[pl.ds(0, _L)] = seg * data_v[pl.ds(64, _L)]
    row_v[pl.ds(_L, _L)] = seg * data_v[pl.ds(80, _L)]
    for k in range(2, 6):
        row_v[pl.ds(k * _L, _L)] = zeros
    pltpu.sync_copy(row_v, out_hbm.at[wid])


def kernel(arg1_1, arg2_1, W):
    vals = jnp.pad(arg1_1.reshape(50), (0, 14))
    data = jnp.concatenate([vals, W])
    idx = jnp.pad(arg2_1, (0, 14))
    return _sc_segsum_outer(data, idx)
